# baseline (speedup 1.0000x reference)
import jax
import jax.numpy as jnp
from jax import lax
from jax.experimental import pallas as pl
from jax.experimental.pallas import tpu as pltpu
from jax.experimental.pallas import tpu_sc as plsc

MAXLEN = 200
EMBED_DIM = 32
BATCH = 4096
TOTAL = BATCH * MAXLEN

NUM_CORES = 2
NUM_SUBCORES = 16
NUM_WORKERS = NUM_CORES * NUM_SUBCORES
ROWS_PER_WORKER = TOTAL // NUM_WORKERS
BATCH_PER_WORKER = BATCH // NUM_WORKERS
CHUNK_B = 4
CHUNK = CHUNK_B * MAXLEN
N_CHUNKS = BATCH_PER_WORKER // CHUNK_B
NBUF = 4


def _gather_body(x_hbm, table_hbm, out_hbm, idx_full, *bufs_and_sems):
    rows_bufs = bufs_and_sems[:NBUF]
    gsems = bufs_and_sems[NBUF:2 * NBUF]
    ssems = bufs_and_sems[2 * NBUF:3 * NBUF]

    wid = lax.axis_index("s") * NUM_CORES + lax.axis_index("c")
    base = wid * ROWS_PER_WORKER
    bbase = wid * BATCH_PER_WORKER

    def start_gather(i, b):
        pltpu.async_copy(table_hbm.at[idx_full.at[pl.ds(i * CHUNK, CHUNK)]],
                         rows_bufs[b], gsems[b])

    def wait_gather(b):
        pltpu.make_async_copy(table_hbm.at[idx_full.at[pl.ds(0, CHUNK)]],
                              rows_bufs[b], gsems[b]).wait()

    def start_store(i, b):
        for j in range(CHUNK_B):
            pltpu.async_copy(rows_bufs[b].at[pl.ds(j * MAXLEN, MAXLEN)],
                             out_hbm.at[bbase + i * CHUNK_B + j], ssems[b])

    def wait_store(b):
        for j in range(CHUNK_B):
            pltpu.make_async_copy(rows_bufs[b].at[pl.ds(j * MAXLEN, MAXLEN)],
                                  out_hbm.at[bbase], ssems[b]).wait()

    pltpu.sync_copy(x_hbm.at[pl.ds(base, ROWS_PER_WORKER)], idx_full)

    for b in range(NBUF):
        start_gather(b, b)

    for i in range(N_CHUNKS):
        b = i % NBUF
        wait_gather(b)
        start_store(i, b)
        if i >= 1:
            pj = i - 1 + NBUF
            if pj < N_CHUNKS:
                pb = (i - 1) % NBUF
                wait_store(pb)
                start_gather(pj, pb)

    for j in range(N_CHUNKS - NBUF, N_CHUNKS):
        wait_store(j % NBUF)


@jax.jit
def kernel(x, token_table, pos_table):
    del pos_table
    xf = x.reshape(-1).astype(jnp.int32)
    mesh = plsc.VectorSubcoreMesh(core_axis_name="c", subcore_axis_name="s")
    return pl.kernel(
        _gather_body,
        out_type=jax.ShapeDtypeStruct((BATCH, MAXLEN, EMBED_DIM), jnp.float32),
        mesh=mesh,
        scratch_types=(
            [pltpu.VMEM((ROWS_PER_WORKER,), jnp.int32)]
            + [pltpu.VMEM((CHUNK, EMBED_DIM), jnp.float32) for _ in range(NBUF)]
            + [pltpu.SemaphoreType.DMA] * (2 * NBUF)
        ),
        compiler_params=pltpu.CompilerParams(use_tc_tiling_on_sc=False),
    )(xf, token_table)


# speedup vs baseline: 1.0003x; 1.0003x over previous
from functools import partial

import jax
import jax.numpy as jnp
from jax import lax
from jax.experimental import pallas as pl
from jax.experimental.layout import Format, Layout
from jax.experimental.pallas import tpu as pltpu
from jax.experimental.pallas import tpu_sc as plsc

MAXLEN = 200
EMBED_DIM = 32
BATCH = 4096
TOTAL = BATCH * MAXLEN

NUM_CORES = 2
NUM_SUBCORES = 16
NUM_WORKERS = NUM_CORES * NUM_SUBCORES
ROWS_PER_WORKER = TOTAL // NUM_WORKERS
BATCH_PER_WORKER = BATCH // NUM_WORKERS
CHUNK_B = 4
CHUNK = CHUNK_B * MAXLEN
N_CHUNKS = BATCH_PER_WORKER // CHUNK_B
NBUF = 4


def _gather_body(x_hbm, table_hbm, out_hbm, idx_full, *bufs_and_sems):
    rows_bufs = bufs_and_sems[:NBUF]
    gsems = bufs_and_sems[NBUF:2 * NBUF]
    ssems = bufs_and_sems[2 * NBUF:3 * NBUF]

    wid = lax.axis_index("s") * NUM_CORES + lax.axis_index("c")
    base = wid * ROWS_PER_WORKER
    bbase = wid * BATCH_PER_WORKER

    def start_gather(i, b):
        pltpu.async_copy(table_hbm.at[idx_full.at[pl.ds(i * CHUNK, CHUNK)]],
                         rows_bufs[b], gsems[b])

    def wait_gather(b):
        pltpu.make_async_copy(table_hbm.at[idx_full.at[pl.ds(0, CHUNK)]],
                              rows_bufs[b], gsems[b]).wait()

    def start_store(i, b):
        for j in range(CHUNK_B):
            pltpu.async_copy(rows_bufs[b].at[pl.ds(j * MAXLEN, MAXLEN)],
                             out_hbm.at[bbase + i * CHUNK_B + j], ssems[b])

    def wait_store(b):
        for j in range(CHUNK_B):
            pltpu.make_async_copy(rows_bufs[b].at[pl.ds(j * MAXLEN, MAXLEN)],
                                  out_hbm.at[bbase], ssems[b]).wait()

    pltpu.sync_copy(x_hbm.at[pl.ds(base, ROWS_PER_WORKER)], idx_full)

    for b in range(NBUF):
        start_gather(b, b)

    for i in range(N_CHUNKS):
        b = i % NBUF
        wait_gather(b)
        start_store(i, b)
        if i >= 1:
            pj = i - 1 + NBUF
            if pj < N_CHUNKS:
                pb = (i - 1) % NBUF
                wait_store(pb)
                start_gather(pj, pb)

    for j in range(N_CHUNKS - NBUF, N_CHUNKS):
        wait_store(j % NBUF)


_OUT_FORMAT = Format(
    Layout(major_to_minor=(0, 1, 2), tiling=()),
    jax.sharding.SingleDeviceSharding(jax.devices()[0]),
)


@partial(jax.jit, out_shardings=_OUT_FORMAT)
def kernel(x, token_table, pos_table):
    del pos_table
    xf = x.reshape(-1).astype(jnp.int32)
    mesh = plsc.VectorSubcoreMesh(core_axis_name="c", subcore_axis_name="s")
    return pl.kernel(
        _gather_body,
        out_type=jax.ShapeDtypeStruct((BATCH, MAXLEN, EMBED_DIM), jnp.float32),
        mesh=mesh,
        scratch_types=(
            [pltpu.VMEM((ROWS_PER_WORKER,), jnp.int32)]
            + [pltpu.VMEM((CHUNK, EMBED_DIM), jnp.float32) for _ in range(NBUF)]
            + [pltpu.SemaphoreType.DMA] * (2 * NBUF)
        ),
        compiler_params=pltpu.CompilerParams(use_tc_tiling_on_sc=False),
    )(xf, token_table)


# speedup vs baseline: 1.0029x; 1.0026x over previous
from functools import partial

import jax
import jax.numpy as jnp
from jax import lax
from jax.experimental import pallas as pl
from jax.experimental.layout import Format, Layout
from jax.experimental.pallas import tpu as pltpu
from jax.experimental.pallas import tpu_sc as plsc

MAXLEN = 200
EMBED_DIM = 32
BATCH = 4096
TOTAL = BATCH * MAXLEN

NUM_CORES = 2
NUM_SUBCORES = 16
NUM_WORKERS = NUM_CORES * NUM_SUBCORES
ROWS_PER_WORKER = TOTAL // NUM_WORKERS
BATCH_PER_WORKER = BATCH // NUM_WORKERS
CHUNK_B = 4
CHUNK = CHUNK_B * MAXLEN
N_CHUNKS = BATCH_PER_WORKER // CHUNK_B
NBUF = 2
NSLOT = 2


def _gather_body(x_hbm, table_hbm, out_hbm, idx_full, spmem, *bufs_and_sems):
    rows_bufs = bufs_and_sems[:NBUF]
    gsems = bufs_and_sems[NBUF:2 * NBUF]
    ssems = bufs_and_sems[2 * NBUF:2 * NBUF + NSLOT]

    cid = lax.axis_index("c")
    tid = lax.axis_index("s")
    wid = tid * NUM_CORES + cid
    base = wid * ROWS_PER_WORKER
    bbase = wid * BATCH_PER_WORKER

    def start_gather(i, b):
        pltpu.async_copy(table_hbm.at[idx_full.at[pl.ds(i * CHUNK, CHUNK)]],
                         rows_bufs[b], gsems[b])

    def wait_gather(b):
        pltpu.make_async_copy(table_hbm.at[idx_full.at[pl.ds(0, CHUNK)]],
                              rows_bufs[b], gsems[b]).wait()

    def start_store(i, s):
        for j in range(CHUNK_B):
            pltpu.async_copy(spmem.at[s, tid].at[pl.ds(j * MAXLEN, MAXLEN)],
                             out_hbm.at[bbase + i * CHUNK_B + j], ssems[s])

    def wait_store(s):
        for j in range(CHUNK_B):
            pltpu.make_async_copy(spmem.at[s, tid].at[pl.ds(j * MAXLEN, MAXLEN)],
                                  out_hbm.at[bbase], ssems[s]).wait()

    pltpu.sync_copy(x_hbm.at[pl.ds(base, ROWS_PER_WORKER)], idx_full)

    for b in range(NBUF):
        start_gather(b, b)

    for i in range(N_CHUNKS):
        b = i % NBUF
        s = i % NSLOT
        wait_gather(b)
        if i >= NSLOT:
            wait_store(s)
        pltpu.sync_copy(rows_bufs[b], spmem.at[s, tid])
        nxt = i + NBUF
        if nxt < N_CHUNKS:
            start_gather(nxt, b)
        start_store(i, s)

    for j in range(max(0, N_CHUNKS - NSLOT), N_CHUNKS):
        wait_store(j % NSLOT)


_OUT_FORMAT = Format(
    Layout(major_to_minor=(0, 1, 2), tiling=()),
    jax.sharding.SingleDeviceSharding(jax.devices()[0]),
)


@partial(jax.jit, out_shardings=_OUT_FORMAT)
def kernel(x, token_table, pos_table):
    del pos_table
    xf = x.reshape(-1).astype(jnp.int32)
    mesh = plsc.VectorSubcoreMesh(core_axis_name="c", subcore_axis_name="s")
    return pl.kernel(
        _gather_body,
        out_type=jax.ShapeDtypeStruct((BATCH, MAXLEN, EMBED_DIM), jnp.float32),
        mesh=mesh,
        scratch_types=(
            [pltpu.VMEM((ROWS_PER_WORKER,), jnp.int32)]
            + [pltpu.VMEM_SHARED((NSLOT, NUM_SUBCORES, CHUNK, EMBED_DIM),
                                 jnp.float32)]
            + [pltpu.VMEM((CHUNK, EMBED_DIM), jnp.float32) for _ in range(NBUF)]
            + [pltpu.SemaphoreType.DMA] * (NBUF + NSLOT)
        ),
        compiler_params=pltpu.CompilerParams(use_tc_tiling_on_sc=False),
    )(xf, token_table)
